# baseline (speedup 1.0000x reference)
import functools

import jax
import jax.numpy as jnp
from jax.experimental import pallas as pl
from jax.experimental.pallas import tpu as pltpu

LANES = 128
CELL = 20
BBLK = 24
H = W = 17


def _mm_kernel(x_ref, w_ref, b_ref, o_ref):
    acc = jnp.dot(x_ref[...], w_ref[...], preferred_element_type=jnp.float32)
    acc = jnp.maximum(acc + b_ref[...], 0.0)
    o_ref[...] = acc.astype(o_ref.dtype)


def _conv_mm(cols, w, b, mblk):
    m, k = cols.shape
    if m % mblk or mblk % 8:
        mblk = m // 8 if (m // 8) % 8 == 0 else 8
    n = w.shape[1]
    return pl.pallas_call(
        _mm_kernel,
        grid=(m // mblk,),
        in_specs=[
            pl.BlockSpec((mblk, k), lambda i: (i, 0)),
            pl.BlockSpec((k, n), lambda i: (0, 0)),
            pl.BlockSpec((1, n), lambda i: (0, 0)),
        ],
        out_specs=pl.BlockSpec((mblk, n), lambda i: (i, 0)),
        out_shape=jax.ShapeDtypeStruct((m, n), jnp.bfloat16),
        compiler_params=pltpu.CompilerParams(
            dimension_semantics=("parallel",)),
    )(cols, w, b)


def _im2col_s2(x_nhwc, k, pad):
    n, h, w, c = x_nhwc.shape
    xp = jnp.pad(x_nhwc, ((0, 0), (pad, pad), (pad, pad), (0, 0)))
    oh = (h + 2 * pad - k) // 2 + 1
    ow = (w + 2 * pad - k) // 2 + 1
    cols = []
    for i in range(k):
        for j in range(k):
            cols.append(xp[:, i:i + 2 * oh:2, j:j + 2 * ow:2, :])
    cols = jnp.concatenate(cols, axis=-1)
    return cols.reshape(n * oh * ow, k * k * c), (oh, ow)


def _tail_kernel(xin_ref, w2_ref, b2_ref, w3_ref, b3_ref, w4_ref, b4_ref,
                 wr3_ref, wr1_ref, o_ref, s_c2, s_in, s_a, s_x, *, bblk,
                 n_res):
    wpb = CELL * bblk
    body = 20 * wpb

    s_c2[...] = jnp.zeros(s_c2.shape, s_c2.dtype)
    s_in[...] = jnp.zeros(s_in.shape, s_in.dtype)
    s_a[...] = jnp.zeros(s_a.shape, s_a.dtype)
    s_x[...] = jnp.zeros(s_x.shape, s_x.dtype)

    for b in range(bblk):
        for y in range(16):
            s_c2[pl.ds((1 + y) * wpb + b * CELL + 1, 16), :] = xin_ref[0, b, y]

    colid = jax.lax.broadcasted_iota(jnp.int32, (wpb, LANES), 0) % CELL
    mask = colid < W
    mask16 = colid < 16

    for y in range(16):
        acc = jnp.zeros((wpb, LANES), jnp.float32)
        for cy in range(3):
            for cx in range(3):
                t = cy * 3 + cx
                acc = acc + jnp.dot(
                    s_c2[pl.ds((y + cy) * wpb + cx, wpb), :],
                    w2_ref[t * 512:(t + 1) * 512, :],
                    preferred_element_type=jnp.float32)
        v = jnp.maximum(acc + b2_ref[...], 0.0)
        s_in[pl.ds((2 + y) * wpb + 2, wpb), :] = jnp.where(mask16, v, 0.0)

    def conv_rows(src, wref, kk, y):
        acc = jnp.zeros((wpb, LANES), jnp.float32)
        for di in range(kk):
            for dj in range(kk):
                t = di * kk + dj
                acc = acc + jnp.dot(
                    src[pl.ds((y + di) * wpb + dj, wpb), :].astype(jnp.bfloat16),
                    wref[t * LANES:(t + 1) * LANES, :],
                    preferred_element_type=jnp.float32)
        return acc

    for y in range(H):
        v = jnp.maximum(conv_rows(s_in, w3_ref, 4, y) + b3_ref[...], 0.0)
        s_a[pl.ds((1 + y) * wpb + 1, wpb), :] = (
            jnp.where(mask, v, 0.0).astype(s_a.dtype))

    for y in range(H):
        v = conv_rows(s_a, w4_ref, 3, y) + b4_ref[...]
        s_x[pl.ds((1 + y) * wpb + 1, wpb), :] = jnp.where(mask, v, 0.0)

    s_in[...] = s_x[...]

    def res_layer(li):
        for y in range(H):
            acc = jnp.zeros((wpb, LANES), jnp.float32)
            for di in range(3):
                for dj in range(3):
                    t = di * 3 + dj
                    acc = acc + jnp.dot(
                        s_a[pl.ds((y + di) * wpb + dj, wpb), :].astype(
                            jnp.bfloat16),
                        wr3_ref[li, t * LANES:(t + 1) * LANES, :],
                        preferred_element_type=jnp.float32)
            h = jnp.maximum(acc, 0.0).astype(jnp.bfloat16)
            d = jnp.dot(h, wr1_ref[li], preferred_element_type=jnp.float32)
            base = (1 + y) * wpb + 1
            s_x[pl.ds(base, wpb), :] = (
                s_x[pl.ds(base, wpb), :] + jnp.where(mask, d, 0.0))

    s_a[...] = jnp.maximum(s_x[...], 0.0)
    res_layer(0)

    i0 = jax.lax.broadcasted_iota(jnp.int32, s_a.shape, 0)
    cc = i0 % CELL
    fresh = (cc == 17) | ((i0 >= 17 * wpb) & (i0 < 18 * wpb) & (cc >= 9))
    s_a[...] = jnp.where(fresh, jnp.maximum(s_x[...], 0.0),
                         jnp.maximum(s_in[...], 0.0))
    res_layer(1)

    for y in range(H):
        o_ref[0, y] = jnp.maximum(
            s_x[pl.ds((1 + y) * wpb + 1, wpb), :], 0.0).astype(o_ref.dtype)


def _tail(x1s2d, w2c, b2, w3, b3, w4, b4, wr3, wr1, bblk):
    nb = x1s2d.shape[0]
    wpb = CELL * bblk
    rows = 20 * wpb + 8
    kern = functools.partial(_tail_kernel, bblk=bblk, n_res=wr3.shape[0])
    return pl.pallas_call(
        kern,
        grid=(nb,),
        in_specs=[
            pl.BlockSpec((1, bblk, 16, 16, 512), lambda i: (i, 0, 0, 0, 0)),
            pl.BlockSpec(w2c.shape, lambda i: (0, 0)),
            pl.BlockSpec(b2.shape, lambda i: (0, 0)),
            pl.BlockSpec(w3.shape, lambda i: (0, 0)),
            pl.BlockSpec(b3.shape, lambda i: (0, 0)),
            pl.BlockSpec(w4.shape, lambda i: (0, 0)),
            pl.BlockSpec(b4.shape, lambda i: (0, 0)),
            pl.BlockSpec(wr3.shape, lambda i: (0, 0, 0)),
            pl.BlockSpec(wr1.shape, lambda i: (0, 0, 0)),
        ],
        out_specs=pl.BlockSpec((1, H, wpb, LANES), lambda i: (i, 0, 0, 0)),
        out_shape=jax.ShapeDtypeStruct((nb, H, wpb, LANES), jnp.bfloat16),
        scratch_shapes=[
            pltpu.VMEM((rows, 512), jnp.bfloat16),
            pltpu.VMEM((rows, LANES), jnp.float32),
            pltpu.VMEM((rows, LANES), jnp.float32),
            pltpu.VMEM((rows, LANES), jnp.float32),
        ],
        compiler_params=pltpu.CompilerParams(
            dimension_semantics=("parallel",),
            vmem_limit_bytes=64 * 1024 * 1024),
    )(x1s2d, w2c, b2, w3, b3, w4, b4, wr3, wr1)


def _pack_w2_cell(w2):
    dmap = [(0, 1), (1, 0), (1, 1), (2, 0)]
    w2v = w2.reshape(4, 4, 64, LANES)
    out = jnp.zeros((3, 3, 2, 2, LANES, LANES), w2.dtype)
    for di in range(4):
        cy, sy = dmap[di]
        for dj in range(4):
            cx, sx = dmap[dj]
            out = out.at[cy, cx, sy, sx, :64, :].set(w2v[di, dj])
    return out.reshape(9 * 512, LANES)


def kernel(w1, b1, w2, b2, w3, b3, w4, b4, wr3, wr1, x):
    B = x.shape[0]
    bf = jnp.bfloat16
    xh = jnp.transpose(x.astype(bf), (0, 2, 3, 1))

    cols, (oh, ow) = _im2col_s2(xh, 4, 1)
    y1 = _conv_mm(cols, w1.astype(bf), b1, mblk=cols.shape[0] // 48)

    nb = B // BBLK
    y1 = y1.reshape(B, oh // 2, 2, ow // 2, 2, LANES)
    y1 = jnp.transpose(y1, (0, 1, 3, 2, 4, 5))
    x1s2d = y1.reshape(nb, BBLK, oh // 2, ow // 2, 4 * LANES)

    w2c = _pack_w2_cell(w2.astype(bf))
    out = _tail(x1s2d, w2c, b2, w3.astype(bf), b3, w4.astype(bf), b4,
                wr3.astype(bf), wr1.astype(bf), BBLK)

    nb = B // BBLK
    out = out.reshape(nb, H, BBLK, CELL, LANES)[:, :, :, :W, :]
    out = jnp.transpose(out, (0, 2, 4, 1, 3)).reshape(B, LANES, H, W)
    return out.astype(jnp.float32)


# speedup vs baseline: 1.0471x; 1.0471x over previous
import functools

import jax
import jax.numpy as jnp
from jax.experimental import pallas as pl
from jax.experimental.pallas import tpu as pltpu

LANES = 128
CELL = 20
BBLK = 16
H = W = 17


def _mm_kernel(x_ref, w_ref, b_ref, o_ref):
    acc = jnp.dot(x_ref[...], w_ref[...], preferred_element_type=jnp.float32)
    acc = jnp.maximum(acc + b_ref[...], 0.0)
    o_ref[...] = acc.astype(o_ref.dtype)


def _conv_mm(cols, w, b, mblk):
    m, k = cols.shape
    if m % mblk or mblk % 8:
        mblk = m // 8 if (m // 8) % 8 == 0 else 8
    n = w.shape[1]
    return pl.pallas_call(
        _mm_kernel,
        grid=(m // mblk,),
        in_specs=[
            pl.BlockSpec((mblk, k), lambda i: (i, 0)),
            pl.BlockSpec((k, n), lambda i: (0, 0)),
            pl.BlockSpec((1, n), lambda i: (0, 0)),
        ],
        out_specs=pl.BlockSpec((mblk, n), lambda i: (i, 0)),
        out_shape=jax.ShapeDtypeStruct((m, n), jnp.bfloat16),
        compiler_params=pltpu.CompilerParams(
            dimension_semantics=("parallel",)),
    )(cols, w, b)


def _im2col_s2(x_nhwc, k, pad):
    n, h, w, c = x_nhwc.shape
    xp = jnp.pad(x_nhwc, ((0, 0), (pad, pad), (pad, pad), (0, 0)))
    oh = (h + 2 * pad - k) // 2 + 1
    ow = (w + 2 * pad - k) // 2 + 1
    cols = []
    for i in range(k):
        for j in range(k):
            cols.append(xp[:, i:i + 2 * oh:2, j:j + 2 * ow:2, :])
    cols = jnp.concatenate(cols, axis=-1)
    return cols.reshape(n * oh * ow, k * k * c), (oh, ow)


def _tail_kernel(xin_ref, w2_ref, b2_ref, w3_ref, b3_ref, w4_ref, b4_ref,
                 wr3_ref, wr1_ref, o_ref, s_c2, s_in, s_a, s_x, *, bblk,
                 n_res):
    wpb = CELL * bblk
    body = 20 * wpb

    s_c2[...] = jnp.zeros(s_c2.shape, s_c2.dtype)
    s_in[...] = jnp.zeros(s_in.shape, s_in.dtype)
    s_a[...] = jnp.zeros(s_a.shape, s_a.dtype)
    s_x[...] = jnp.zeros(s_x.shape, s_x.dtype)

    for b in range(bblk):
        for y in range(16):
            s_c2[pl.ds((1 + y) * wpb + b * CELL + 1, 16), :] = xin_ref[0, b, y]

    colid = jax.lax.broadcasted_iota(jnp.int32, (wpb, LANES), 0) % CELL
    mask = colid < W
    mask16 = colid < 16

    for y in range(16):
        acc = jnp.zeros((wpb, LANES), jnp.float32)
        for cy in range(3):
            for cx in range(3):
                t = cy * 3 + cx
                acc = acc + jnp.dot(
                    s_c2[pl.ds((y + cy) * wpb + cx, wpb), :],
                    w2_ref[t * 512:(t + 1) * 512, :],
                    preferred_element_type=jnp.float32)
        v = jnp.maximum(acc + b2_ref[...], 0.0)
        s_in[pl.ds((2 + y) * wpb + 2, wpb), :] = jnp.where(mask16, v, 0.0)

    def conv_rows(src, wref, kk, y):
        acc = jnp.zeros((wpb, LANES), jnp.float32)
        for di in range(kk):
            for dj in range(kk):
                t = di * kk + dj
                acc = acc + jnp.dot(
                    src[pl.ds((y + di) * wpb + dj, wpb), :].astype(jnp.bfloat16),
                    wref[t * LANES:(t + 1) * LANES, :],
                    preferred_element_type=jnp.float32)
        return acc

    for y in range(H):
        v = jnp.maximum(conv_rows(s_in, w3_ref, 4, y) + b3_ref[...], 0.0)
        s_a[pl.ds((1 + y) * wpb + 1, wpb), :] = (
            jnp.where(mask, v, 0.0).astype(s_a.dtype))

    for y in range(H):
        v = conv_rows(s_a, w4_ref, 3, y) + b4_ref[...]
        s_x[pl.ds((1 + y) * wpb + 1, wpb), :] = jnp.where(mask, v, 0.0)

    s_in[...] = s_x[...]

    def res_layer(li):
        for y in range(H):
            acc = jnp.zeros((wpb, LANES), jnp.float32)
            for di in range(3):
                for dj in range(3):
                    t = di * 3 + dj
                    acc = acc + jnp.dot(
                        s_a[pl.ds((y + di) * wpb + dj, wpb), :].astype(
                            jnp.bfloat16),
                        wr3_ref[li, t * LANES:(t + 1) * LANES, :],
                        preferred_element_type=jnp.float32)
            h = jnp.maximum(acc, 0.0).astype(jnp.bfloat16)
            d = jnp.dot(h, wr1_ref[li], preferred_element_type=jnp.float32)
            base = (1 + y) * wpb + 1
            s_x[pl.ds(base, wpb), :] = (
                s_x[pl.ds(base, wpb), :] + jnp.where(mask, d, 0.0))

    s_a[...] = jnp.maximum(s_x[...], 0.0)
    res_layer(0)

    i0 = jax.lax.broadcasted_iota(jnp.int32, s_a.shape, 0)
    cc = i0 % CELL
    fresh = (cc == 17) | ((i0 >= 17 * wpb) & (i0 < 18 * wpb) & (cc >= 9))
    s_a[...] = jnp.where(fresh, jnp.maximum(s_x[...], 0.0),
                         jnp.maximum(s_in[...], 0.0))
    res_layer(1)

    for y in range(H):
        o_ref[0, y] = jnp.maximum(
            s_x[pl.ds((1 + y) * wpb + 1, wpb), :], 0.0).astype(o_ref.dtype)


def _tail(x1s2d, w2c, b2, w3, b3, w4, b4, wr3, wr1, bblk):
    nb = x1s2d.shape[0]
    wpb = CELL * bblk
    rows = 20 * wpb + 8
    kern = functools.partial(_tail_kernel, bblk=bblk, n_res=wr3.shape[0])
    return pl.pallas_call(
        kern,
        grid=(nb,),
        in_specs=[
            pl.BlockSpec((1, bblk, 16, 16, 512), lambda i: (i, 0, 0, 0, 0)),
            pl.BlockSpec(w2c.shape, lambda i: (0, 0)),
            pl.BlockSpec(b2.shape, lambda i: (0, 0)),
            pl.BlockSpec(w3.shape, lambda i: (0, 0)),
            pl.BlockSpec(b3.shape, lambda i: (0, 0)),
            pl.BlockSpec(w4.shape, lambda i: (0, 0)),
            pl.BlockSpec(b4.shape, lambda i: (0, 0)),
            pl.BlockSpec(wr3.shape, lambda i: (0, 0, 0)),
            pl.BlockSpec(wr1.shape, lambda i: (0, 0, 0)),
        ],
        out_specs=pl.BlockSpec((1, H, wpb, LANES), lambda i: (i, 0, 0, 0)),
        out_shape=jax.ShapeDtypeStruct((nb, H, wpb, LANES), jnp.bfloat16),
        scratch_shapes=[
            pltpu.VMEM((rows, 512), jnp.bfloat16),
            pltpu.VMEM((rows, LANES), jnp.float32),
            pltpu.VMEM((rows, LANES), jnp.float32),
            pltpu.VMEM((rows, LANES), jnp.float32),
        ],
        compiler_params=pltpu.CompilerParams(
            dimension_semantics=("parallel",),
            vmem_limit_bytes=64 * 1024 * 1024),
    )(x1s2d, w2c, b2, w3, b3, w4, b4, wr3, wr1)


def _pack_w2_cell(w2):
    dmap = [(0, 1), (1, 0), (1, 1), (2, 0)]
    w2v = w2.reshape(4, 4, 64, LANES)
    out = jnp.zeros((3, 3, 2, 2, LANES, LANES), w2.dtype)
    for di in range(4):
        cy, sy = dmap[di]
        for dj in range(4):
            cx, sx = dmap[dj]
            out = out.at[cy, cx, sy, sx, :64, :].set(w2v[di, dj])
    return out.reshape(9 * 512, LANES)


def kernel(w1, b1, w2, b2, w3, b3, w4, b4, wr3, wr1, x):
    B = x.shape[0]
    bf = jnp.bfloat16
    xh = jnp.transpose(x.astype(bf), (0, 2, 3, 1))

    cols, (oh, ow) = _im2col_s2(xh, 4, 1)
    y1 = _conv_mm(cols, w1.astype(bf), b1, mblk=cols.shape[0] // 48)

    nb = B // BBLK
    y1 = y1.reshape(B, oh // 2, 2, ow // 2, 2, LANES)
    y1 = jnp.transpose(y1, (0, 1, 3, 2, 4, 5))
    x1s2d = y1.reshape(nb, BBLK, oh // 2, ow // 2, 4 * LANES)

    w2c = _pack_w2_cell(w2.astype(bf))
    out = _tail(x1s2d, w2c, b2, w3.astype(bf), b3, w4.astype(bf), b4,
                wr3.astype(bf), wr1.astype(bf), BBLK)

    nb = B // BBLK
    out = out.reshape(nb, H, BBLK, CELL, LANES)[:, :, :, :W, :]
    out = jnp.transpose(out, (0, 2, 4, 1, 3)).reshape(B, LANES, H, W)
    return out.astype(jnp.float32)
